# baseline (speedup 1.0000x reference)
import functools

import jax
import jax.numpy as jnp
from jax import lax
from jax.experimental import pallas as pl
from jax.experimental.pallas import tpu as pltpu
from jax.experimental.pallas import tpu_sc as plsc

N = 50000
H = 64
HW = 32
E = 800000
NP = 50176
NS = 16
EPT = E // NS
BATCH = 128
NB = 393
EHALF = E // 2
CPT = EHALF // NS
CNB = 196
EPAD = 800640
CW = 16
ACC_ROWS = 50304
DUMMY = ACC_ROWS - 1
ZPT = 3144
OPT = 3136
RB = 1024
GRID = NP // RB

_mesh = plsc.VectorSubcoreMesh(
    core_axis_name="c", subcore_axis_name="s", num_cores=2, num_subcores=NS)
_sc_params = pltpu.CompilerParams(use_tc_tiling_on_sc=False)


def _zero_buf(buf, nrows, width):
    z = jnp.zeros((16,), jnp.float32)

    @pl.loop(0, nrows)
    def _(i):
        for j in range(width // 16):
            buf[i, pl.ds(16 * j, 16)] = z


@functools.partial(
    pl.kernel,
    out_type=[jax.ShapeDtypeStruct((2 * NP, HW), jnp.float32)] * 3,
    mesh=_mesh,
    scratch_types=[
        pltpu.VMEM_SHARED((ACC_ROWS, HW), jnp.float32),
        pltpu.VMEM((3, BATCH), jnp.int32),
        pltpu.VMEM((3, BATCH), jnp.int32),
        pltpu.VMEM((3, BATCH), jnp.int32),
        pltpu.VMEM((3, BATCH, HW), jnp.float32),
        pltpu.VMEM((BATCH, HW), jnp.float32),
        [pltpu.SemaphoreType.DMA] * 3,
        [pltpu.SemaphoreType.DMA] * 3,
        [pltpu.SemaphoreType.DMA] * 3,
    ],
    compiler_params=_sc_params,
)
def _segsum3(te, tt, src_c, dst_c, src_r, dst_r, src_d, dst_d,
             out_c, out_r, out_d, acc, srcv, draw, dst2, rows, zbuf,
             sem_i, sem_g, sem_s):
    c = lax.axis_index("c")
    s = lax.axis_index("s")
    coff = c * NP

    def adjust_src(b):
        for jj in range(BATCH // 16):
            sv = srcv[b, pl.ds(16 * jj, 16)]
            srcv[b, pl.ds(16 * jj, 16)] = sv + coff

    def adjust_dst(b, kk):
        for jj in range(BATCH // 16):
            d = draw[b, pl.ds(16 * jj, 16)]
            pos = lax.iota(jnp.int32, 16) + (kk * BATCH + 16 * jj)
            dst2[b, pl.ds(16 * jj, 16)] = jnp.where(pos < EPT, d, DUMMY)

    def scatter_drain(b):
        pltpu.make_async_copy(rows.at[b], acc.at[dst2.at[b]],
                              sem_s[b]).wait()

    _zero_buf(zbuf, BATCH, HW)
    r0 = s * ZPT
    zd = [pltpu.async_copy(zbuf, acc.at[pl.ds(r0 + q * BATCH, BATCH)],
                           sem_g[0]) for q in range(24)]
    zd.append(pltpu.async_copy(zbuf.at[pl.ds(0, 72)],
                               acc.at[pl.ds(r0 + 24 * BATCH, 72)], sem_g[0]))
    for d in zd:
        d.wait()
    plsc.subcore_barrier()

    for table, src, dst, out in ((te, src_c, dst_c, out_c),
                                 (tt, src_r, dst_r, out_r),
                                 (te, src_d, dst_d, out_d)):

        def idx_issue(kk, b):
            base = pl.multiple_of(s * EPT + kk * BATCH, 8)
            pltpu.async_copy(src.at[pl.ds(base, BATCH)], srcv.at[b],
                             sem_i[b])
            pltpu.async_copy(dst.at[pl.ds(base, BATCH)], draw.at[b],
                             sem_i[b])

        def idx_drain(b):
            pltpu.make_async_copy(src.at[pl.ds(0, BATCH)], srcv.at[b],
                                  sem_i[b]).wait()
            pltpu.make_async_copy(dst.at[pl.ds(0, BATCH)], draw.at[b],
                                  sem_i[b]).wait()

        def gather_issue(b):
            pltpu.async_copy(table.at[srcv.at[b]], rows.at[b], sem_g[b])

        def gather_drain(b):
            pltpu.make_async_copy(table.at[srcv.at[b]], rows.at[b],
                                  sem_g[b]).wait()

        for b in range(3):
            idx_issue(b, b)
        idx_drain(0)
        adjust_src(0)
        gather_issue(0)

        @pl.loop(0, NB // 3)
        def _(k):
            for b in range(3):
                kk = 3 * k + b
                b1 = (b + 1) % 3

                @pl.when(kk >= 2)
                def _():
                    scatter_drain(b1)

                @pl.when(kk + 1 < NB)
                def _():
                    idx_drain(b1)
                    adjust_src(b1)
                    gather_issue(b1)

                adjust_dst(b, kk)
                gather_drain(b)
                pltpu.async_copy(rows.at[b], acc.at[dst2.at[b]], sem_s[b],
                                 add=True)

                @pl.when(kk + 3 < NB)
                def _():
                    idx_issue(kk + 3, b)

        scatter_drain((NB - 2) % 3)
        scatter_drain((NB - 1) % 3)
        plsc.subcore_barrier()

        o0 = s * OPT
        rz = []
        for q in range(24):
            rb = rows.at[q % 3]
            pltpu.sync_copy(acc.at[pl.ds(o0 + q * BATCH, BATCH)], rb)
            rz.append(pltpu.async_copy(
                zbuf, acc.at[pl.ds(o0 + q * BATCH, BATCH)], sem_g[0]))
            pltpu.sync_copy(rb, out.at[pl.ds(coff + o0 + q * BATCH, BATCH)])
        tb = rows.at[0].at[pl.ds(0, 64)]
        pltpu.sync_copy(acc.at[pl.ds(o0 + 24 * BATCH, 64)], tb)
        rz.append(pltpu.async_copy(
            zbuf.at[pl.ds(0, 64)], acc.at[pl.ds(o0 + 24 * BATCH, 64)],
            sem_g[0]))
        pltpu.sync_copy(tb, out.at[pl.ds(coff + o0 + 24 * BATCH, 64)])
        for d in rz:
            d.wait()
        plsc.subcore_barrier()


@functools.partial(
    pl.kernel,
    out_type=[jax.ShapeDtypeStruct((2 * NP, CW), jnp.float32)] * 3,
    mesh=_mesh,
    scratch_types=[
        pltpu.VMEM_SHARED((ACC_ROWS, CW), jnp.float32),
        pltpu.VMEM((2, BATCH), jnp.int32),
        pltpu.VMEM((2, BATCH), jnp.int32),
        pltpu.VMEM((BATCH, CW), jnp.float32),
        pltpu.VMEM((BATCH, CW), jnp.float32),
        pltpu.VMEM((BATCH, CW), jnp.float32),
        [pltpu.SemaphoreType.DMA] * 2,
        [pltpu.SemaphoreType.DMA] * 2,
    ],
    compiler_params=_sc_params,
)
def _counts3(dst_c, dst_r, dst_d, out_c, out_r, out_d, acc, draw, dst2,
             ones, zbuf, obuf, sem_i, sem_s):
    c = lax.axis_index("c")
    s = lax.axis_index("s")
    ebase0 = c * EHALF + s * CPT
    one = jnp.ones((16,), jnp.float32)

    @pl.loop(0, BATCH)
    def _(i):
        ones[i, pl.ds(0, 16)] = one

    def adjust_dst(b, kk):
        for jj in range(BATCH // 16):
            d = draw[b, pl.ds(16 * jj, 16)]
            pos = lax.iota(jnp.int32, 16) + (kk * BATCH + 16 * jj)
            dst2[b, pl.ds(16 * jj, 16)] = jnp.where(pos < CPT, d, DUMMY)

    def scatter_drain(b):
        pltpu.make_async_copy(ones, acc.at[dst2.at[b]], sem_s[b]).wait()

    _zero_buf(zbuf, BATCH, CW)
    r0 = s * ZPT
    zd = [pltpu.async_copy(zbuf, acc.at[pl.ds(r0 + q * BATCH, BATCH)],
                           sem_i[0]) for q in range(24)]
    zd.append(pltpu.async_copy(zbuf.at[pl.ds(0, 72)],
                               acc.at[pl.ds(r0 + 24 * BATCH, 72)], sem_i[0]))
    for d in zd:
        d.wait()
    plsc.subcore_barrier()

    for dst, out in ((dst_c, out_c), (dst_r, out_r), (dst_d, out_d)):

        def idx_issue(kk, b):
            base = pl.multiple_of(ebase0 + kk * BATCH, 8)
            pltpu.async_copy(dst.at[pl.ds(base, BATCH)], draw.at[b],
                             sem_i[b])

        def idx_drain(b):
            pltpu.make_async_copy(dst.at[pl.ds(0, BATCH)], draw.at[b],
                                  sem_i[b]).wait()

        for b in range(2):
            idx_issue(b, b)

        @pl.loop(0, CNB // 2)
        def _(k):
            for b in range(2):
                kk = 2 * k + b

                @pl.when(kk >= 2)
                def _():
                    scatter_drain(b)

                idx_drain(b)
                adjust_dst(b, kk)
                pltpu.async_copy(ones, acc.at[dst2.at[b]], sem_s[b],
                                 add=True)

                @pl.when(kk + 2 < CNB)
                def _():
                    idx_issue(kk + 2, b)

        scatter_drain(0)
        scatter_drain(1)
        plsc.subcore_barrier()

        o0 = s * OPT
        rz = []
        for q in range(24):
            pltpu.sync_copy(acc.at[pl.ds(o0 + q * BATCH, BATCH)], obuf)
            rz.append(pltpu.async_copy(
                zbuf, acc.at[pl.ds(o0 + q * BATCH, BATCH)], sem_i[0]))
            pltpu.sync_copy(obuf,
                            out.at[pl.ds(c * NP + o0 + q * BATCH, BATCH)])
        tb = obuf.at[pl.ds(0, 64)]
        pltpu.sync_copy(acc.at[pl.ds(o0 + 24 * BATCH, 64)], tb)
        rz.append(pltpu.async_copy(
            zbuf.at[pl.ds(0, 64)], acc.at[pl.ds(o0 + 24 * BATCH, 64)],
            sem_i[0]))
        pltpu.sync_copy(tb, out.at[pl.ds(c * NP + o0 + 24 * BATCH, 64)])
        for d in rz:
            d.wait()
        plsc.subcore_barrier()


def _row_mask(i):
    rid = lax.broadcasted_iota(jnp.int32, (RB, 1), 0) + i * RB
    return rid < N


def _dg(a, w):
    return lax.dot_general(a, w, (((1,), (1,)), ((), ())),
                           preferred_element_type=jnp.float32)


def _proj_body(xe, xt, we, be, wt, bt, oe, ot):
    i = pl.program_id(1)
    m = _row_mask(i)
    oe[...] = jnp.where(m, _dg(xe[...], we[...]) + be[0], 0.0)
    ot[...] = jnp.where(m, _dg(xt[...], wt[...]) + bt[0], 0.0)


def _stats_rows(o):
    row8 = lax.broadcasted_iota(jnp.int32, (1, 8, HW), 1)
    su = jnp.sum(o, axis=0)[None, None, :]
    sq = jnp.sum(o * o, axis=0)[None, None, :]
    return jnp.where(row8 == 0, su, jnp.where(row8 == 1, sq, 0.0))


def _d2a_body(scl, scr, srl, srr, sdl, sdr, cca, ccb, cra, crb, cda, cdb,
              xel, xer, xtl, xtr, wlee, wlte, wlet, wree, wrte, wret,
              blee, blte, blet, oe_, ot_, ste_, stt_):
    i = pl.program_id(1)
    m = _row_mask(i)

    def mean(l_ref, r_ref, ca, cb):
        cnt = jnp.maximum(ca[...][:, 0:1] + cb[...][:, 0:1], 1.0)
        return jnp.concatenate([l_ref[...], r_ref[...]], axis=1) / cnt

    xe = jnp.concatenate([xel[...], xer[...]], axis=1)
    xt = jnp.concatenate([xtl[...], xtr[...]], axis=1)
    oe = (_dg(mean(scl, scr, cca, ccb), wlee[...])
          + _dg(mean(srl, srr, cra, crb), wlte[...])
          + _dg(xe, wree[...]) + _dg(xe, wrte[...])
          + blee[0] + blte[0])
    ot = (_dg(mean(sdl, sdr, cda, cdb), wlet[...]) + _dg(xt, wret[...])
          + blet[0])
    oe = jnp.where(m, oe, 0.0)
    ot = jnp.where(m, ot, 0.0)
    oe_[...] = oe
    ot_[...] = ot
    se = _stats_rows(oe)
    st = _stats_rows(ot)

    @pl.when(i == 0)
    def _():
        ste_[...] = se
        stt_[...] = st

    @pl.when(i > 0)
    def _():
        ste_[...] += se
        stt_[...] += st


def _d2b_body(oe_, ot_, ste, stt, ge, be, gt, bt, xe_, xt_):
    i = pl.program_id(1)
    m = _row_mask(i)

    def bn(o_ref, st_ref, g_ref, b_ref):
        mu = st_ref[0, 0, :] / float(N)
        var = st_ref[0, 1, :] / float(N) - mu * mu
        inv = lax.rsqrt(var + 1e-5)
        y = (o_ref[...] - mu) * inv * g_ref[0] + b_ref[0]
        return jnp.where(m, jnp.maximum(y, 0.0), 0.0)

    xe_[...] = bn(oe_, ste, ge, be)
    xt_[...] = bn(ot_, stt, gt, bt)


_f32 = jnp.float32


def _stk():
    return pl.BlockSpec((RB, HW), lambda p, i: (p * GRID + i, 0))


def _b32():
    return pl.BlockSpec((1, 1, HW), lambda p, i: (p, 0, 0))


def _st32():
    return pl.BlockSpec((1, 8, HW), lambda p, i: (p, 0, 0))


def _proj(xe, xt, we, be, wt, bt):
    return pl.pallas_call(
        _proj_body,
        grid=(2, GRID),
        in_specs=[pl.BlockSpec((RB, 4), lambda p, i: (i, 0)),
                  pl.BlockSpec((RB, 4), lambda p, i: (i, 0)),
                  pl.BlockSpec((HW, 4), lambda p, i: (p, 0)),
                  _b32(),
                  pl.BlockSpec((HW, 4), lambda p, i: (p, 0)),
                  _b32()],
        out_specs=[_stk(), _stk()],
        out_shape=[jax.ShapeDtypeStruct((2 * NP, HW), _f32)] * 2,
    )(xe, xt, we, be, wt, bt)


def _halves(arr_w):
    lo = pl.BlockSpec((RB, arr_w), lambda p, i: (i, 0))
    hi = pl.BlockSpec((RB, arr_w), lambda p, i: (GRID + i, 0))
    return lo, hi


def _d2a(s_c, s_r, s_d, c_c, c_r, c_d, xe, xt, w):
    (wlee, blee, wree, wlet, blet, wret, wlte, blte, wrte) = w
    sl, sh = _halves(HW)
    cl, ch = _halves(CW)
    wl = pl.BlockSpec((HW, H), lambda p, i: (p, 0))
    return pl.pallas_call(
        _d2a_body,
        grid=(2, GRID),
        in_specs=[sl, sh, sl, sh, sl, sh,
                  cl, ch, cl, ch, cl, ch,
                  sl, sh, sl, sh]
        + [wl] * 6 + [_b32()] * 3,
        out_specs=[_stk(), _stk(), _st32(), _st32()],
        out_shape=[jax.ShapeDtypeStruct((2 * NP, HW), _f32)] * 2
        + [jax.ShapeDtypeStruct((2, 8, HW), _f32)] * 2,
    )(s_c, s_c, s_r, s_r, s_d, s_d, c_c, c_c, c_r, c_r, c_d, c_d,
      xe, xe, xt, xt, wlee, wlte, wlet, wree, wrte, wret, blee, blte, blet)


def _d2b(oe, ot, ste, stt, ge, be, gt, bt):
    return pl.pallas_call(
        _d2b_body,
        grid=(2, GRID),
        in_specs=[_stk(), _stk(), _st32(), _st32()] + [_b32()] * 4,
        out_specs=[_stk(), _stk()],
        out_shape=[jax.ShapeDtypeStruct((2 * NP, HW), _f32)] * 2,
    )(oe, ot, ste, stt, ge, be, gt, bt)


def kernel(x_epsilon, x_theta, ei_comm, ei_det, ei_rev, proj_eps_W,
           proj_eps_b, proj_th_W, proj_th_b, l0_ee_Wl, l0_ee_bl, l0_ee_Wr,
           l0_et_Wl, l0_et_bl, l0_et_Wr, l0_te_Wl, l0_te_bl, l0_te_Wr,
           l0_bn_eps_g, l0_bn_eps_b, l0_bn_th_g, l0_bn_th_b, l1_ee_Wl,
           l1_ee_bl, l1_ee_Wr, l1_et_Wl, l1_et_bl, l1_et_Wr, l1_te_Wl,
           l1_te_bl, l1_te_Wr, l1_bn_eps_g, l1_bn_eps_b, l1_bn_th_g,
           l1_bn_th_b, l2_ee_Wl, l2_ee_bl, l2_ee_Wr, l2_et_Wl, l2_et_bl,
           l2_et_Wr, l2_te_Wl, l2_te_bl, l2_te_Wr, l2_bn_eps_g, l2_bn_eps_b,
           l2_bn_th_g, l2_bn_th_b):
    row = lambda v: v.reshape(2, 1, HW)
    xep = jnp.pad(x_epsilon, ((0, NP - N), (0, 0)))
    xtp = jnp.pad(x_theta, ((0, NP - N), (0, 0)))
    padi = lambda a: jnp.pad(a, (0, EPAD - E))
    src_c, dst_c = padi(ei_comm[0]), padi(ei_comm[1])
    src_d, dst_d = padi(ei_det[0]), padi(ei_det[1])
    src_r, dst_r = padi(ei_rev[0]), padi(ei_rev[1])

    cnt_c, cnt_r, cnt_d = _counts3(dst_c, dst_r, dst_d)

    xe, xt = _proj(xep, xtp, proj_eps_W, row(proj_eps_b),
                   proj_th_W, row(proj_th_b))

    layers = [
        (l0_ee_Wl, l0_ee_bl, l0_ee_Wr, l0_et_Wl, l0_et_bl, l0_et_Wr,
         l0_te_Wl, l0_te_bl, l0_te_Wr, l0_bn_eps_g, l0_bn_eps_b,
         l0_bn_th_g, l0_bn_th_b),
        (l1_ee_Wl, l1_ee_bl, l1_ee_Wr, l1_et_Wl, l1_et_bl, l1_et_Wr,
         l1_te_Wl, l1_te_bl, l1_te_Wr, l1_bn_eps_g, l1_bn_eps_b,
         l1_bn_th_g, l1_bn_th_b),
        (l2_ee_Wl, l2_ee_bl, l2_ee_Wr, l2_et_Wl, l2_et_bl, l2_et_Wr,
         l2_te_Wl, l2_te_bl, l2_te_Wr, l2_bn_eps_g, l2_bn_eps_b,
         l2_bn_th_g, l2_bn_th_b),
    ]
    for (wlee, blee, wree, wlet, blet, wret, wlte, blte, wrte,
         gep, bep, gth, bth) in layers:
        s_c, s_r, s_d = _segsum3(xe, xt, src_c, dst_c, src_r, dst_r,
                                 src_d, dst_d)
        oe, ot, ste, stt = _d2a(
            s_c, s_r, s_d, cnt_c, cnt_r, cnt_d, xe, xt,
            (wlee, row(blee), wree, wlet, row(blet), wret,
             wlte, row(blte), wrte))
        xe, xt = _d2b(oe, ot, ste, stt, row(gep), row(bep),
                      row(gth), row(bth))
    return (jnp.concatenate([xe[:N], xe[NP:NP + N]], axis=1),
            jnp.concatenate([xt[:N], xt[NP:NP + N]], axis=1))


# speedup vs baseline: 1.1443x; 1.1443x over previous
import functools

import jax
import jax.numpy as jnp
from jax import lax
from jax.experimental import pallas as pl
from jax.experimental.pallas import tpu as pltpu
from jax.experimental.pallas import tpu_sc as plsc

N = 50000
H = 64
HW = 32
E = 800000
NP = 50176
NS = 16
EPT = E // NS
BATCH = 128
NB4 = 396
EHALF = E // 2
CPT = EHALF // NS
CNB = 196
EPAD = 800768
CW = 16
ACC_ROWS = 50304
DUMMY = ACC_ROWS - 1
ZPT = 3144
OPT = 3136
RB = 1024
GRID = NP // RB

_mesh = plsc.VectorSubcoreMesh(
    core_axis_name="c", subcore_axis_name="s", num_cores=2, num_subcores=NS)
_sc_params = pltpu.CompilerParams(use_tc_tiling_on_sc=False)


def _zero_buf(buf, nrows, width):
    z = jnp.zeros((16,), jnp.float32)

    @pl.loop(0, nrows)
    def _(i):
        for j in range(width // 16):
            buf[i, pl.ds(16 * j, 16)] = z


@functools.partial(
    pl.kernel,
    out_type=jax.ShapeDtypeStruct((2 * NP, HW), jnp.float32),
    mesh=_mesh,
    scratch_types=[
        pltpu.VMEM_SHARED((ACC_ROWS, HW), jnp.float32),
        pltpu.VMEM((4, BATCH), jnp.int32),
        pltpu.VMEM((4, BATCH), jnp.int32),
        pltpu.VMEM((4, BATCH), jnp.int32),
        pltpu.VMEM((4, BATCH, HW), jnp.float32),
        [pltpu.SemaphoreType.DMA] * 4,
        [pltpu.SemaphoreType.DMA] * 4,
        [pltpu.SemaphoreType.DMA] * 4,
    ],
    compiler_params=_sc_params,
)
def _segsum(table, src, dst, out, acc, srcv, draw, dst2, rows, sem_i, sem_g,
            sem_s):
    c = lax.axis_index("c")
    s = lax.axis_index("s")
    coff = c * NP

    def idx_issue(kk, b):
        base = pl.multiple_of(s * EPT + kk * BATCH, 8)
        pltpu.async_copy(src.at[pl.ds(base, BATCH)], srcv.at[b], sem_i[b])
        pltpu.async_copy(dst.at[pl.ds(base, BATCH)], draw.at[b], sem_i[b])

    def idx_drain(b):
        pltpu.make_async_copy(src.at[pl.ds(0, BATCH)], srcv.at[b],
                              sem_i[b]).wait()
        pltpu.make_async_copy(dst.at[pl.ds(0, BATCH)], draw.at[b],
                              sem_i[b]).wait()

    def adjust_src(b):
        for jj in range(BATCH // 16):
            sv = srcv[b, pl.ds(16 * jj, 16)]
            srcv[b, pl.ds(16 * jj, 16)] = sv + coff

    def adjust_dst(b, kk):
        for jj in range(BATCH // 16):
            d = draw[b, pl.ds(16 * jj, 16)]
            pos = lax.iota(jnp.int32, 16) + (kk * BATCH + 16 * jj)
            dst2[b, pl.ds(16 * jj, 16)] = jnp.where(pos < EPT, d, DUMMY)

    def gather_issue(b):
        pltpu.async_copy(table.at[srcv.at[b]], rows.at[b], sem_g[b])

    def gather_drain(b):
        pltpu.make_async_copy(table.at[srcv.at[b]], rows.at[b],
                              sem_g[b]).wait()

    def scatter_drain(b):
        pltpu.make_async_copy(rows.at[b], acc.at[dst2.at[b]],
                              sem_s[b]).wait()

    _zero_buf(rows.at[0], BATCH, HW)
    r0 = s * ZPT
    zd = [pltpu.async_copy(rows.at[0], acc.at[pl.ds(r0 + q * BATCH, BATCH)],
                           sem_g[0]) for q in range(24)]
    zd.append(pltpu.async_copy(rows.at[0].at[pl.ds(0, 72)],
                               acc.at[pl.ds(r0 + 24 * BATCH, 72)], sem_g[0]))
    for d in zd:
        d.wait()
    plsc.subcore_barrier()

    for b in range(4):
        idx_issue(b, b)
    for b in range(2):
        idx_drain(b)
        adjust_src(b)
        gather_issue(b)

    @pl.loop(0, NB4 // 4)
    def _(k):
        for b in range(4):
            kk = 4 * k + b
            b2 = (b + 2) % 4

            @pl.when(kk >= 2)
            def _():
                scatter_drain(b2)

            @pl.when(kk + 2 < NB4)
            def _():
                idx_drain(b2)
                adjust_src(b2)
                gather_issue(b2)

            adjust_dst(b, kk)
            gather_drain(b)
            pltpu.async_copy(rows.at[b], acc.at[dst2.at[b]], sem_s[b],
                             add=True)

            @pl.when(kk + 4 < NB4)
            def _():
                idx_issue(kk + 4, b)

    scatter_drain((NB4 - 2) % 4)
    scatter_drain((NB4 - 1) % 4)
    plsc.subcore_barrier()

    o0 = s * OPT
    for q in range(24):
        rb = rows.at[q % 4]
        pltpu.sync_copy(acc.at[pl.ds(o0 + q * BATCH, BATCH)], rb)
        pltpu.sync_copy(rb, out.at[pl.ds(coff + o0 + q * BATCH, BATCH)])
    tb = rows.at[0].at[pl.ds(0, 64)]
    pltpu.sync_copy(acc.at[pl.ds(o0 + 24 * BATCH, 64)], tb)
    pltpu.sync_copy(tb, out.at[pl.ds(coff + o0 + 24 * BATCH, 64)])


@functools.partial(
    pl.kernel,
    out_type=jax.ShapeDtypeStruct((2 * NP, CW), jnp.float32),
    mesh=_mesh,
    scratch_types=[
        pltpu.VMEM_SHARED((ACC_ROWS, CW), jnp.float32),
        pltpu.VMEM((2, BATCH), jnp.int32),
        pltpu.VMEM((2, BATCH), jnp.int32),
        pltpu.VMEM((BATCH, CW), jnp.float32),
        pltpu.VMEM((BATCH, CW), jnp.float32),
        [pltpu.SemaphoreType.DMA] * 2,
        [pltpu.SemaphoreType.DMA] * 2,
    ],
    compiler_params=_sc_params,
)
def _counts(dst, out, acc, draw, dst2, ones, zbuf, sem_i, sem_s):
    c = lax.axis_index("c")
    s = lax.axis_index("s")
    ebase0 = c * EHALF + s * CPT
    one = jnp.ones((16,), jnp.float32)

    @pl.loop(0, BATCH)
    def _(i):
        ones[i, pl.ds(0, 16)] = one

    def idx_issue(kk, b):
        base = pl.multiple_of(ebase0 + kk * BATCH, 8)
        pltpu.async_copy(dst.at[pl.ds(base, BATCH)], draw.at[b], sem_i[b])

    def idx_drain(b):
        pltpu.make_async_copy(dst.at[pl.ds(0, BATCH)], draw.at[b],
                              sem_i[b]).wait()

    def adjust_dst(b, kk):
        for jj in range(BATCH // 16):
            d = draw[b, pl.ds(16 * jj, 16)]
            pos = lax.iota(jnp.int32, 16) + (kk * BATCH + 16 * jj)
            dst2[b, pl.ds(16 * jj, 16)] = jnp.where(pos < CPT, d, DUMMY)

    def scatter_drain(b):
        pltpu.make_async_copy(ones, acc.at[dst2.at[b]], sem_s[b]).wait()

    _zero_buf(zbuf, BATCH, CW)
    r0 = s * ZPT
    zd = [pltpu.async_copy(zbuf, acc.at[pl.ds(r0 + q * BATCH, BATCH)],
                           sem_i[0]) for q in range(24)]
    zd.append(pltpu.async_copy(zbuf.at[pl.ds(0, 72)],
                               acc.at[pl.ds(r0 + 24 * BATCH, 72)], sem_i[0]))
    for d in zd:
        d.wait()
    plsc.subcore_barrier()

    for b in range(2):
        idx_issue(b, b)

    @pl.loop(0, CNB // 2)
    def _(k):
        for b in range(2):
            kk = 2 * k + b

            @pl.when(kk >= 2)
            def _():
                scatter_drain(b)

            idx_drain(b)
            adjust_dst(b, kk)
            pltpu.async_copy(ones, acc.at[dst2.at[b]], sem_s[b], add=True)

            @pl.when(kk + 2 < CNB)
            def _():
                idx_issue(kk + 2, b)

    scatter_drain(0)
    scatter_drain(1)
    plsc.subcore_barrier()

    o0 = s * OPT
    for q in range(24):
        pltpu.sync_copy(acc.at[pl.ds(o0 + q * BATCH, BATCH)], zbuf)
        pltpu.sync_copy(zbuf, out.at[pl.ds(c * NP + o0 + q * BATCH, BATCH)])
    tb = zbuf.at[pl.ds(0, 64)]
    pltpu.sync_copy(acc.at[pl.ds(o0 + 24 * BATCH, 64)], tb)
    pltpu.sync_copy(tb, out.at[pl.ds(c * NP + o0 + 24 * BATCH, 64)])


def _row_mask(i):
    rid = lax.broadcasted_iota(jnp.int32, (RB, 1), 0) + i * RB
    return rid < N


def _dg(a, w):
    return lax.dot_general(a, w, (((1,), (1,)), ((), ())),
                           preferred_element_type=jnp.float32)


def _proj_body(xe, xt, we, be, wt, bt, oe, ot):
    i = pl.program_id(1)
    m = _row_mask(i)
    oe[...] = jnp.where(m, _dg(xe[...], we[...]) + be[0], 0.0)
    ot[...] = jnp.where(m, _dg(xt[...], wt[...]) + bt[0], 0.0)


def _stats_rows(o):
    row8 = lax.broadcasted_iota(jnp.int32, (1, 8, HW), 1)
    su = jnp.sum(o, axis=0)[None, None, :]
    sq = jnp.sum(o * o, axis=0)[None, None, :]
    return jnp.where(row8 == 0, su, jnp.where(row8 == 1, sq, 0.0))


def _d2a_body(scl, scr, srl, srr, sdl, sdr, cca, ccb, cra, crb, cda, cdb,
              xel, xer, xtl, xtr, wlee, wlte, wlet, wree, wrte, wret,
              blee, blte, blet, oe_, ot_, ste_, stt_):
    i = pl.program_id(1)
    m = _row_mask(i)

    def mean(l_ref, r_ref, ca, cb):
        cnt = jnp.maximum(ca[...][:, 0:1] + cb[...][:, 0:1], 1.0)
        return jnp.concatenate([l_ref[...], r_ref[...]], axis=1) / cnt

    xe = jnp.concatenate([xel[...], xer[...]], axis=1)
    xt = jnp.concatenate([xtl[...], xtr[...]], axis=1)
    oe = (_dg(mean(scl, scr, cca, ccb), wlee[...])
          + _dg(mean(srl, srr, cra, crb), wlte[...])
          + _dg(xe, wree[...]) + _dg(xe, wrte[...])
          + blee[0] + blte[0])
    ot = (_dg(mean(sdl, sdr, cda, cdb), wlet[...]) + _dg(xt, wret[...])
          + blet[0])
    oe = jnp.where(m, oe, 0.0)
    ot = jnp.where(m, ot, 0.0)
    oe_[...] = oe
    ot_[...] = ot
    se = _stats_rows(oe)
    st = _stats_rows(ot)

    @pl.when(i == 0)
    def _():
        ste_[...] = se
        stt_[...] = st

    @pl.when(i > 0)
    def _():
        ste_[...] += se
        stt_[...] += st


def _d2b_body(oe_, ot_, ste, stt, ge, be, gt, bt, xe_, xt_):
    i = pl.program_id(1)
    m = _row_mask(i)

    def bn(o_ref, st_ref, g_ref, b_ref):
        mu = st_ref[0, 0, :] / float(N)
        var = st_ref[0, 1, :] / float(N) - mu * mu
        inv = lax.rsqrt(var + 1e-5)
        y = (o_ref[...] - mu) * inv * g_ref[0] + b_ref[0]
        return jnp.where(m, jnp.maximum(y, 0.0), 0.0)

    xe_[...] = bn(oe_, ste, ge, be)
    xt_[...] = bn(ot_, stt, gt, bt)


_f32 = jnp.float32


def _stk():
    return pl.BlockSpec((RB, HW), lambda p, i: (p * GRID + i, 0))


def _b32():
    return pl.BlockSpec((1, 1, HW), lambda p, i: (p, 0, 0))


def _st32():
    return pl.BlockSpec((1, 8, HW), lambda p, i: (p, 0, 0))


def _proj(xe, xt, we, be, wt, bt):
    return pl.pallas_call(
        _proj_body,
        grid=(2, GRID),
        in_specs=[pl.BlockSpec((RB, 4), lambda p, i: (i, 0)),
                  pl.BlockSpec((RB, 4), lambda p, i: (i, 0)),
                  pl.BlockSpec((HW, 4), lambda p, i: (p, 0)),
                  _b32(),
                  pl.BlockSpec((HW, 4), lambda p, i: (p, 0)),
                  _b32()],
        out_specs=[_stk(), _stk()],
        out_shape=[jax.ShapeDtypeStruct((2 * NP, HW), _f32)] * 2,
    )(xe, xt, we, be, wt, bt)


def _halves(arr_w):
    lo = pl.BlockSpec((RB, arr_w), lambda p, i: (i, 0))
    hi = pl.BlockSpec((RB, arr_w), lambda p, i: (GRID + i, 0))
    return lo, hi


def _d2a(s_c, s_r, s_d, c_c, c_r, c_d, xe, xt, w):
    (wlee, blee, wree, wlet, blet, wret, wlte, blte, wrte) = w
    sl, sh = _halves(HW)
    cl, ch = _halves(CW)
    wl = pl.BlockSpec((HW, H), lambda p, i: (p, 0))
    return pl.pallas_call(
        _d2a_body,
        grid=(2, GRID),
        in_specs=[sl, sh, sl, sh, sl, sh,
                  cl, ch, cl, ch, cl, ch,
                  sl, sh, sl, sh]
        + [wl] * 6 + [_b32()] * 3,
        out_specs=[_stk(), _stk(), _st32(), _st32()],
        out_shape=[jax.ShapeDtypeStruct((2 * NP, HW), _f32)] * 2
        + [jax.ShapeDtypeStruct((2, 8, HW), _f32)] * 2,
    )(s_c, s_c, s_r, s_r, s_d, s_d, c_c, c_c, c_r, c_r, c_d, c_d,
      xe, xe, xt, xt, wlee, wlte, wlet, wree, wrte, wret, blee, blte, blet)


def _d2b(oe, ot, ste, stt, ge, be, gt, bt):
    return pl.pallas_call(
        _d2b_body,
        grid=(2, GRID),
        in_specs=[_stk(), _stk(), _st32(), _st32()] + [_b32()] * 4,
        out_specs=[_stk(), _stk()],
        out_shape=[jax.ShapeDtypeStruct((2 * NP, HW), _f32)] * 2,
    )(oe, ot, ste, stt, ge, be, gt, bt)


def kernel(x_epsilon, x_theta, ei_comm, ei_det, ei_rev, proj_eps_W,
           proj_eps_b, proj_th_W, proj_th_b, l0_ee_Wl, l0_ee_bl, l0_ee_Wr,
           l0_et_Wl, l0_et_bl, l0_et_Wr, l0_te_Wl, l0_te_bl, l0_te_Wr,
           l0_bn_eps_g, l0_bn_eps_b, l0_bn_th_g, l0_bn_th_b, l1_ee_Wl,
           l1_ee_bl, l1_ee_Wr, l1_et_Wl, l1_et_bl, l1_et_Wr, l1_te_Wl,
           l1_te_bl, l1_te_Wr, l1_bn_eps_g, l1_bn_eps_b, l1_bn_th_g,
           l1_bn_th_b, l2_ee_Wl, l2_ee_bl, l2_ee_Wr, l2_et_Wl, l2_et_bl,
           l2_et_Wr, l2_te_Wl, l2_te_bl, l2_te_Wr, l2_bn_eps_g, l2_bn_eps_b,
           l2_bn_th_g, l2_bn_th_b):
    row = lambda v: v.reshape(2, 1, HW)
    xep = jnp.pad(x_epsilon, ((0, NP - N), (0, 0)))
    xtp = jnp.pad(x_theta, ((0, NP - N), (0, 0)))
    padi = lambda a: jnp.pad(a, (0, EPAD - E))
    src_c, dst_c = padi(ei_comm[0]), padi(ei_comm[1])
    src_d, dst_d = padi(ei_det[0]), padi(ei_det[1])
    src_r, dst_r = padi(ei_rev[0]), padi(ei_rev[1])

    cnt_c = _counts(dst_c)
    cnt_r = _counts(dst_r)
    cnt_d = _counts(dst_d)

    xe, xt = _proj(xep, xtp, proj_eps_W, row(proj_eps_b),
                   proj_th_W, row(proj_th_b))

    layers = [
        (l0_ee_Wl, l0_ee_bl, l0_ee_Wr, l0_et_Wl, l0_et_bl, l0_et_Wr,
         l0_te_Wl, l0_te_bl, l0_te_Wr, l0_bn_eps_g, l0_bn_eps_b,
         l0_bn_th_g, l0_bn_th_b),
        (l1_ee_Wl, l1_ee_bl, l1_ee_Wr, l1_et_Wl, l1_et_bl, l1_et_Wr,
         l1_te_Wl, l1_te_bl, l1_te_Wr, l1_bn_eps_g, l1_bn_eps_b,
         l1_bn_th_g, l1_bn_th_b),
        (l2_ee_Wl, l2_ee_bl, l2_ee_Wr, l2_et_Wl, l2_et_bl, l2_et_Wr,
         l2_te_Wl, l2_te_bl, l2_te_Wr, l2_bn_eps_g, l2_bn_eps_b,
         l2_bn_th_g, l2_bn_th_b),
    ]
    for (wlee, blee, wree, wlet, blet, wret, wlte, blte, wrte,
         gep, bep, gth, bth) in layers:
        s_c = _segsum(xe, src_c, dst_c)
        s_r = _segsum(xt, src_r, dst_r)
        s_d = _segsum(xe, src_d, dst_d)
        oe, ot, ste, stt = _d2a(
            s_c, s_r, s_d, cnt_c, cnt_r, cnt_d, xe, xt,
            (wlee, row(blee), wree, wlet, row(blet), wret,
             wlte, row(blte), wrte))
        xe, xt = _d2b(oe, ot, ste, stt, row(gep), row(bep),
                      row(gth), row(bth))
    return (jnp.concatenate([xe[:N], xe[NP:NP + N]], axis=1),
            jnp.concatenate([xt[:N], xt[NP:NP + N]], axis=1))


# speedup vs baseline: 1.1812x; 1.0323x over previous
import functools

import jax
import jax.numpy as jnp
from jax import lax
from jax.experimental import pallas as pl
from jax.experimental.pallas import tpu as pltpu
from jax.experimental.pallas import tpu_sc as plsc

N = 50000
H = 64
HW = 32
E = 800000
NP = 50176
NS = 16
EPT = E // NS
BATCH = 128
NB4 = 396
EHALF = E // 2
CPT = EHALF // NS
CNB = 196
EPAD = 800768
CW = 16
ACC_ROWS = 50304
DUMMY = ACC_ROWS - 1
ZPT = 3144
OPT = 3136
RB = 1792
GRID = NP // RB

_mesh = plsc.VectorSubcoreMesh(
    core_axis_name="c", subcore_axis_name="s", num_cores=2, num_subcores=NS)
_sc_params = pltpu.CompilerParams(use_tc_tiling_on_sc=False)


def _zero_buf(buf, nrows, width):
    z = jnp.zeros((16,), jnp.float32)

    @pl.loop(0, nrows)
    def _(i):
        for j in range(width // 16):
            buf[i, pl.ds(16 * j, 16)] = z


@functools.partial(
    pl.kernel,
    out_type=jax.ShapeDtypeStruct((2 * NP, HW), jnp.float32),
    mesh=_mesh,
    scratch_types=[
        pltpu.VMEM_SHARED((ACC_ROWS, HW), jnp.float32),
        pltpu.VMEM((4, BATCH), jnp.int32),
        pltpu.VMEM((4, BATCH), jnp.int32),
        pltpu.VMEM((4, BATCH), jnp.int32),
        pltpu.VMEM((4, BATCH, HW), jnp.float32),
        [pltpu.SemaphoreType.DMA] * 4,
        [pltpu.SemaphoreType.DMA] * 4,
        [pltpu.SemaphoreType.DMA] * 4,
    ],
    compiler_params=_sc_params,
)
def _segsum(table, src, dst, out, acc, srcv, draw, dst2, rows, sem_i, sem_g,
            sem_s):
    c = lax.axis_index("c")
    s = lax.axis_index("s")
    coff = c * NP

    def idx_issue(kk, b):
        base = pl.multiple_of(s * EPT + kk * BATCH, 8)
        pltpu.async_copy(src.at[pl.ds(base, BATCH)], srcv.at[b], sem_i[b])
        pltpu.async_copy(dst.at[pl.ds(base, BATCH)], draw.at[b], sem_i[b])

    def idx_drain(b):
        pltpu.make_async_copy(src.at[pl.ds(0, BATCH)], srcv.at[b],
                              sem_i[b]).wait()
        pltpu.make_async_copy(dst.at[pl.ds(0, BATCH)], draw.at[b],
                              sem_i[b]).wait()

    def adjust_src(b):
        for jj in range(BATCH // 16):
            sv = srcv[b, pl.ds(16 * jj, 16)]
            srcv[b, pl.ds(16 * jj, 16)] = sv + coff

    def adjust_dst(b, kk):
        for jj in range(BATCH // 16):
            d = draw[b, pl.ds(16 * jj, 16)]
            pos = lax.iota(jnp.int32, 16) + (kk * BATCH + 16 * jj)
            dst2[b, pl.ds(16 * jj, 16)] = jnp.where(pos < EPT, d, DUMMY)

    def gather_issue(b):
        pltpu.async_copy(table.at[srcv.at[b]], rows.at[b], sem_g[b])

    def gather_drain(b):
        pltpu.make_async_copy(table.at[srcv.at[b]], rows.at[b],
                              sem_g[b]).wait()

    def scatter_drain(b):
        pltpu.make_async_copy(rows.at[b], acc.at[dst2.at[b]],
                              sem_s[b]).wait()

    _zero_buf(rows.at[0], BATCH, HW)
    r0 = s * ZPT
    zd = [pltpu.async_copy(rows.at[0], acc.at[pl.ds(r0 + q * BATCH, BATCH)],
                           sem_g[0]) for q in range(24)]
    zd.append(pltpu.async_copy(rows.at[0].at[pl.ds(0, 72)],
                               acc.at[pl.ds(r0 + 24 * BATCH, 72)], sem_g[0]))
    for d in zd:
        d.wait()
    plsc.subcore_barrier()

    for b in range(4):
        idx_issue(b, b)
    for b in range(2):
        idx_drain(b)
        adjust_src(b)
        gather_issue(b)

    @pl.loop(0, NB4 // 4)
    def _(k):
        for b in range(4):
            kk = 4 * k + b
            b2 = (b + 2) % 4

            @pl.when(kk >= 2)
            def _():
                scatter_drain(b2)

            @pl.when(kk + 2 < NB4)
            def _():
                idx_drain(b2)
                adjust_src(b2)
                gather_issue(b2)

            adjust_dst(b, kk)
            gather_drain(b)
            pltpu.async_copy(rows.at[b], acc.at[dst2.at[b]], sem_s[b],
                             add=True)

            @pl.when(kk + 4 < NB4)
            def _():
                idx_issue(kk + 4, b)

    scatter_drain((NB4 - 2) % 4)
    scatter_drain((NB4 - 1) % 4)
    plsc.subcore_barrier()

    o0 = s * OPT
    for q in range(24):
        rb = rows.at[q % 4]
        pltpu.sync_copy(acc.at[pl.ds(o0 + q * BATCH, BATCH)], rb)
        pltpu.sync_copy(rb, out.at[pl.ds(coff + o0 + q * BATCH, BATCH)])
    tb = rows.at[0].at[pl.ds(0, 64)]
    pltpu.sync_copy(acc.at[pl.ds(o0 + 24 * BATCH, 64)], tb)
    pltpu.sync_copy(tb, out.at[pl.ds(coff + o0 + 24 * BATCH, 64)])


@functools.partial(
    pl.kernel,
    out_type=jax.ShapeDtypeStruct((2 * NP, CW), jnp.float32),
    mesh=_mesh,
    scratch_types=[
        pltpu.VMEM_SHARED((ACC_ROWS, CW), jnp.float32),
        pltpu.VMEM((2, BATCH), jnp.int32),
        pltpu.VMEM((2, BATCH), jnp.int32),
        pltpu.VMEM((BATCH, CW), jnp.float32),
        pltpu.VMEM((BATCH, CW), jnp.float32),
        [pltpu.SemaphoreType.DMA] * 2,
        [pltpu.SemaphoreType.DMA] * 2,
    ],
    compiler_params=_sc_params,
)
def _counts(dst, out, acc, draw, dst2, ones, zbuf, sem_i, sem_s):
    c = lax.axis_index("c")
    s = lax.axis_index("s")
    ebase0 = c * EHALF + s * CPT
    one = jnp.ones((16,), jnp.float32)

    @pl.loop(0, BATCH)
    def _(i):
        ones[i, pl.ds(0, 16)] = one

    def idx_issue(kk, b):
        base = pl.multiple_of(ebase0 + kk * BATCH, 8)
        pltpu.async_copy(dst.at[pl.ds(base, BATCH)], draw.at[b], sem_i[b])

    def idx_drain(b):
        pltpu.make_async_copy(dst.at[pl.ds(0, BATCH)], draw.at[b],
                              sem_i[b]).wait()

    def adjust_dst(b, kk):
        for jj in range(BATCH // 16):
            d = draw[b, pl.ds(16 * jj, 16)]
            pos = lax.iota(jnp.int32, 16) + (kk * BATCH + 16 * jj)
            dst2[b, pl.ds(16 * jj, 16)] = jnp.where(pos < CPT, d, DUMMY)

    def scatter_drain(b):
        pltpu.make_async_copy(ones, acc.at[dst2.at[b]], sem_s[b]).wait()

    _zero_buf(zbuf, BATCH, CW)
    r0 = s * ZPT
    zd = [pltpu.async_copy(zbuf, acc.at[pl.ds(r0 + q * BATCH, BATCH)],
                           sem_i[0]) for q in range(24)]
    zd.append(pltpu.async_copy(zbuf.at[pl.ds(0, 72)],
                               acc.at[pl.ds(r0 + 24 * BATCH, 72)], sem_i[0]))
    for d in zd:
        d.wait()
    plsc.subcore_barrier()

    for b in range(2):
        idx_issue(b, b)

    @pl.loop(0, CNB // 2)
    def _(k):
        for b in range(2):
            kk = 2 * k + b

            @pl.when(kk >= 2)
            def _():
                scatter_drain(b)

            idx_drain(b)
            adjust_dst(b, kk)
            pltpu.async_copy(ones, acc.at[dst2.at[b]], sem_s[b], add=True)

            @pl.when(kk + 2 < CNB)
            def _():
                idx_issue(kk + 2, b)

    scatter_drain(0)
    scatter_drain(1)
    plsc.subcore_barrier()

    o0 = s * OPT
    for q in range(24):
        pltpu.sync_copy(acc.at[pl.ds(o0 + q * BATCH, BATCH)], zbuf)
        pltpu.sync_copy(zbuf, out.at[pl.ds(c * NP + o0 + q * BATCH, BATCH)])
    tb = zbuf.at[pl.ds(0, 64)]
    pltpu.sync_copy(acc.at[pl.ds(o0 + 24 * BATCH, 64)], tb)
    pltpu.sync_copy(tb, out.at[pl.ds(c * NP + o0 + 24 * BATCH, 64)])


def _row_mask(i):
    rid = lax.broadcasted_iota(jnp.int32, (RB, 1), 0) + i * RB
    return rid < N


def _dg(a, w):
    return lax.dot_general(a, w, (((1,), (1,)), ((), ())),
                           preferred_element_type=jnp.float32)


def _proj_body(xe, xt, we, be, wt, bt, oe, ot):
    i = pl.program_id(1)
    m = _row_mask(i)
    oe[...] = jnp.where(m, _dg(xe[...], we[...]) + be[0], 0.0)
    ot[...] = jnp.where(m, _dg(xt[...], wt[...]) + bt[0], 0.0)


def _stats_rows(o):
    row8 = lax.broadcasted_iota(jnp.int32, (1, 8, HW), 1)
    su = jnp.sum(o, axis=0)[None, None, :]
    sq = jnp.sum(o * o, axis=0)[None, None, :]
    return jnp.where(row8 == 0, su, jnp.where(row8 == 1, sq, 0.0))


def _d2a_body(scl, scr, srl, srr, sdl, sdr, cca, ccb, cra, crb, cda, cdb,
              xel, xer, xtl, xtr, wlee, wlte, wlet, wree, wrte, wret,
              blee, blte, blet, oe_, ot_, ste_, stt_):
    i = pl.program_id(1)
    m = _row_mask(i)

    def mean(l_ref, r_ref, ca, cb):
        cnt = jnp.maximum(ca[...][:, 0:1] + cb[...][:, 0:1], 1.0)
        return jnp.concatenate([l_ref[...], r_ref[...]], axis=1) / cnt

    xe = jnp.concatenate([xel[...], xer[...]], axis=1)
    xt = jnp.concatenate([xtl[...], xtr[...]], axis=1)
    ae = jnp.concatenate([mean(scl, scr, cca, ccb),
                          mean(srl, srr, cra, crb), xe], axis=1)
    we = jnp.concatenate([wlee[...], wlte[...], wree[...] + wrte[...]],
                         axis=1)
    oe = _dg(ae, we) + blee[0] + blte[0]
    at = jnp.concatenate([mean(sdl, sdr, cda, cdb), xt], axis=1)
    wt = jnp.concatenate([wlet[...], wret[...]], axis=1)
    ot = _dg(at, wt) + blet[0]
    oe = jnp.where(m, oe, 0.0)
    ot = jnp.where(m, ot, 0.0)
    oe_[...] = oe
    ot_[...] = ot
    se = _stats_rows(oe)
    st = _stats_rows(ot)

    @pl.when(i == 0)
    def _():
        ste_[...] = se
        stt_[...] = st

    @pl.when(i > 0)
    def _():
        ste_[...] += se
        stt_[...] += st


def _d2b_body(oe_, ot_, ste, stt, ge, be, gt, bt, xe_, xt_):
    i = pl.program_id(1)
    m = _row_mask(i)

    def bn(o_ref, st_ref, g_ref, b_ref):
        mu = st_ref[0, 0, :] / float(N)
        var = st_ref[0, 1, :] / float(N) - mu * mu
        inv = lax.rsqrt(var + 1e-5)
        y = (o_ref[...] - mu) * inv * g_ref[0] + b_ref[0]
        return jnp.where(m, jnp.maximum(y, 0.0), 0.0)

    xe_[...] = bn(oe_, ste, ge, be)
    xt_[...] = bn(ot_, stt, gt, bt)


_f32 = jnp.float32


def _stk():
    return pl.BlockSpec((RB, HW), lambda p, i: (p * GRID + i, 0))


def _b32():
    return pl.BlockSpec((1, 1, HW), lambda p, i: (p, 0, 0))


def _st32():
    return pl.BlockSpec((1, 8, HW), lambda p, i: (p, 0, 0))


def _proj(xe, xt, we, be, wt, bt):
    return pl.pallas_call(
        _proj_body,
        grid=(2, GRID),
        in_specs=[pl.BlockSpec((RB, 4), lambda p, i: (i, 0)),
                  pl.BlockSpec((RB, 4), lambda p, i: (i, 0)),
                  pl.BlockSpec((HW, 4), lambda p, i: (p, 0)),
                  _b32(),
                  pl.BlockSpec((HW, 4), lambda p, i: (p, 0)),
                  _b32()],
        out_specs=[_stk(), _stk()],
        out_shape=[jax.ShapeDtypeStruct((2 * NP, HW), _f32)] * 2,
    )(xe, xt, we, be, wt, bt)


def _halves(arr_w):
    lo = pl.BlockSpec((RB, arr_w), lambda p, i: (i, 0))
    hi = pl.BlockSpec((RB, arr_w), lambda p, i: (GRID + i, 0))
    return lo, hi


def _d2a(s_c, s_r, s_d, c_c, c_r, c_d, xe, xt, w):
    (wlee, blee, wree, wlet, blet, wret, wlte, blte, wrte) = w
    sl, sh = _halves(HW)
    cl, ch = _halves(CW)
    wl = pl.BlockSpec((HW, H), lambda p, i: (p, 0))
    return pl.pallas_call(
        _d2a_body,
        grid=(2, GRID),
        in_specs=[sl, sh, sl, sh, sl, sh,
                  cl, ch, cl, ch, cl, ch,
                  sl, sh, sl, sh]
        + [wl] * 6 + [_b32()] * 3,
        out_specs=[_stk(), _stk(), _st32(), _st32()],
        out_shape=[jax.ShapeDtypeStruct((2 * NP, HW), _f32)] * 2
        + [jax.ShapeDtypeStruct((2, 8, HW), _f32)] * 2,
    )(s_c, s_c, s_r, s_r, s_d, s_d, c_c, c_c, c_r, c_r, c_d, c_d,
      xe, xe, xt, xt, wlee, wlte, wlet, wree, wrte, wret, blee, blte, blet)


def _d2b(oe, ot, ste, stt, ge, be, gt, bt):
    return pl.pallas_call(
        _d2b_body,
        grid=(2, GRID),
        in_specs=[_stk(), _stk(), _st32(), _st32()] + [_b32()] * 4,
        out_specs=[_stk(), _stk()],
        out_shape=[jax.ShapeDtypeStruct((2 * NP, HW), _f32)] * 2,
    )(oe, ot, ste, stt, ge, be, gt, bt)


def kernel(x_epsilon, x_theta, ei_comm, ei_det, ei_rev, proj_eps_W,
           proj_eps_b, proj_th_W, proj_th_b, l0_ee_Wl, l0_ee_bl, l0_ee_Wr,
           l0_et_Wl, l0_et_bl, l0_et_Wr, l0_te_Wl, l0_te_bl, l0_te_Wr,
           l0_bn_eps_g, l0_bn_eps_b, l0_bn_th_g, l0_bn_th_b, l1_ee_Wl,
           l1_ee_bl, l1_ee_Wr, l1_et_Wl, l1_et_bl, l1_et_Wr, l1_te_Wl,
           l1_te_bl, l1_te_Wr, l1_bn_eps_g, l1_bn_eps_b, l1_bn_th_g,
           l1_bn_th_b, l2_ee_Wl, l2_ee_bl, l2_ee_Wr, l2_et_Wl, l2_et_bl,
           l2_et_Wr, l2_te_Wl, l2_te_bl, l2_te_Wr, l2_bn_eps_g, l2_bn_eps_b,
           l2_bn_th_g, l2_bn_th_b):
    row = lambda v: v.reshape(2, 1, HW)
    xep = jnp.pad(x_epsilon, ((0, NP - N), (0, 0)))
    xtp = jnp.pad(x_theta, ((0, NP - N), (0, 0)))
    padi = lambda a: jnp.pad(a, (0, EPAD - E))
    src_c, dst_c = padi(ei_comm[0]), padi(ei_comm[1])
    src_d, dst_d = padi(ei_det[0]), padi(ei_det[1])
    src_r, dst_r = padi(ei_rev[0]), padi(ei_rev[1])

    cnt_c = _counts(dst_c)
    cnt_r = _counts(dst_r)
    cnt_d = _counts(dst_d)

    xe, xt = _proj(xep, xtp, proj_eps_W, row(proj_eps_b),
                   proj_th_W, row(proj_th_b))

    layers = [
        (l0_ee_Wl, l0_ee_bl, l0_ee_Wr, l0_et_Wl, l0_et_bl, l0_et_Wr,
         l0_te_Wl, l0_te_bl, l0_te_Wr, l0_bn_eps_g, l0_bn_eps_b,
         l0_bn_th_g, l0_bn_th_b),
        (l1_ee_Wl, l1_ee_bl, l1_ee_Wr, l1_et_Wl, l1_et_bl, l1_et_Wr,
         l1_te_Wl, l1_te_bl, l1_te_Wr, l1_bn_eps_g, l1_bn_eps_b,
         l1_bn_th_g, l1_bn_th_b),
        (l2_ee_Wl, l2_ee_bl, l2_ee_Wr, l2_et_Wl, l2_et_bl, l2_et_Wr,
         l2_te_Wl, l2_te_bl, l2_te_Wr, l2_bn_eps_g, l2_bn_eps_b,
         l2_bn_th_g, l2_bn_th_b),
    ]
    for (wlee, blee, wree, wlet, blet, wret, wlte, blte, wrte,
         gep, bep, gth, bth) in layers:
        s_c = _segsum(xe, src_c, dst_c)
        s_r = _segsum(xt, src_r, dst_r)
        s_d = _segsum(xe, src_d, dst_d)
        oe, ot, ste, stt = _d2a(
            s_c, s_r, s_d, cnt_c, cnt_r, cnt_d, xe, xt,
            (wlee, row(blee), wree, wlet, row(blet), wret,
             wlte, row(blte), wrte))
        xe, xt = _d2b(oe, ot, ste, stt, row(gep), row(bep),
                      row(gth), row(bth))
    return (jnp.concatenate([xe[:N], xe[NP:NP + N]], axis=1),
            jnp.concatenate([xt[:N], xt[NP:NP + N]], axis=1))
